# baseline (speedup 1.0000x reference)
import functools

import jax
import jax.numpy as jnp
from jax import lax
from jax.experimental import pallas as pl
from jax.experimental.pallas import tpu as pltpu
from jax.experimental.pallas import tpu_sc as plsc

_EPS = 1e-5
_CHUNK = 128


def _sc_gather(table, idx2d, n_rows, d):
    info = plsc.get_sparse_core_info()
    nc, ns = info.num_cores, info.num_subcores
    nw = nc * ns
    n_chunks = idx2d.shape[0]
    chunks_per_w = n_chunks // nw
    rows_per_w = chunks_per_w * _CHUNK
    mesh = plsc.VectorSubcoreMesh(core_axis_name="c", subcore_axis_name="s")

    @functools.partial(
        pl.kernel,
        mesh=mesh,
        out_type=jax.ShapeDtypeStruct((n_rows, d), jnp.float32),
        scratch_types=[
            pltpu.VMEM((chunks_per_w, _CHUNK), jnp.int32),
            pltpu.VMEM((rows_per_w, d), jnp.float32),
            pltpu.SemaphoreType.DMA,
        ],
    )
    def k(table_hbm, idx_hbm, out_hbm, idx_v, rows_v, sem):
        wid = lax.axis_index("s") * nc + lax.axis_index("c")
        pltpu.sync_copy(idx_hbm.at[pl.ds(wid * chunks_per_w, chunks_per_w)], idx_v)
        copies = []
        for j in range(chunks_per_w):
            copies.append(
                pltpu.async_copy(
                    table_hbm.at[idx_v.at[j]],
                    rows_v.at[pl.ds(j * _CHUNK, _CHUNK)],
                    sem,
                )
            )
        for c in copies:
            c.wait()
        pltpu.sync_copy(rows_v, out_hbm.at[pl.ds(wid * rows_per_w, rows_per_w)])

    return k(table, idx2d)


_R = 1024


def _tc_manual(batch, seq, embed, hidden,
               e_hbm, w_ref, b_ref, pos_hbm, g_ref, bt_ref, o_hbm,
               pos_v, e_v, obuf, sem_pos, sem_e, sem_st):
    n_rows = batch * seq
    s_blks = seq // _R
    n_blk = n_rows // _R

    pos_cp = []
    for s in range(s_blks):
        c = pltpu.make_async_copy(
            pos_hbm.at[pl.ds(s * _R, _R)], pos_v.at[pl.ds(s * _R, _R)],
            sem_pos.at[s])
        c.start()
        pos_cp.append(c)

    order = [(s, b) for s in range(s_blks) for b in range(batch)]
    e_cp = {}
    for i, (s, b) in enumerate(order):
        row0 = b * seq + s * _R
        c = pltpu.make_async_copy(
            e_hbm.at[pl.ds(row0, _R)], e_v.at[pl.ds(i * _R, _R)], sem_e.at[i])
        c.start()
        e_cp[i] = c

    w = w_ref[...]
    bias = b_ref[...]
    gam = g_ref[...]
    bet = bt_ref[...]

    st_cp = {}
    for i, (s, b) in enumerate(order):
        if b == 0:
            pos_cp[s].wait()
        e_cp[i].wait()
        if i >= 2:
            st_cp[i - 2].wait()
        h = jax.lax.dot_general(
            e_v[pl.ds(i * _R, _R), :], w,
            dimension_numbers=(((1,), (0,)), ((), ())),
            preferred_element_type=jnp.float32,
        )
        h = h + bias + pos_v[pl.ds(s * _R, _R), :]
        mean = jnp.mean(h, axis=-1, keepdims=True)
        cen = h - mean
        var = jnp.mean(cen * cen, axis=-1, keepdims=True)
        obuf[i % 2] = cen * jax.lax.rsqrt(var + _EPS) * gam + bet
        row0 = b * seq + s * _R
        c = pltpu.make_async_copy(
            obuf.at[i % 2], o_hbm.at[pl.ds(row0, _R)], sem_st.at[i])
        c.start()
        st_cp[i] = c
    st_cp[n_blk - 2].wait()
    st_cp[n_blk - 1].wait()


def kernel(x, tok_embed1, W2, b2, pos_embed, gamma, beta):
    batch, seq = x.shape
    vocab, embed = tok_embed1.shape
    hidden = W2.shape[1]
    n_rows = batch * seq

    idx2d = x.reshape(n_rows // _CHUNK, _CHUNK)
    e = _sc_gather(tok_embed1, idx2d, n_rows, embed)

    s_blks = seq // _R
    n_blk = n_rows // _R

    body = functools.partial(_tc_manual, batch, seq, embed, hidden)
    out = pl.pallas_call(
        body,
        in_specs=[
            pl.BlockSpec(memory_space=pl.ANY),
            pl.BlockSpec((embed, hidden), lambda: (0, 0)),
            pl.BlockSpec((1, hidden), lambda: (0, 0)),
            pl.BlockSpec(memory_space=pl.ANY),
            pl.BlockSpec((1, hidden), lambda: (0, 0)),
            pl.BlockSpec((1, hidden), lambda: (0, 0)),
        ],
        out_specs=pl.BlockSpec(memory_space=pl.ANY),
        out_shape=jax.ShapeDtypeStruct((n_rows, hidden), jnp.float32),
        scratch_shapes=[
            pltpu.VMEM((seq, hidden), jnp.float32),
            pltpu.VMEM((n_rows, embed), jnp.float32),
            pltpu.VMEM((2, _R, hidden), jnp.float32),
            pltpu.SemaphoreType.DMA((s_blks,)),
            pltpu.SemaphoreType.DMA((n_blk,)),
            pltpu.SemaphoreType.DMA((n_blk,)),
        ],
    )(
        e,
        W2,
        b2.reshape(1, hidden),
        pos_embed,
        gamma.reshape(1, hidden),
        beta.reshape(1, hidden),
    )
    return out.reshape(batch, seq, hidden)
